# Initial kernel scaffold; baseline (speedup 1.0000x reference)
import jax
import jax.numpy as jnp
from jax.experimental import pallas as pl


def kernel(inputs, codebooks):
    raise NotImplementedError("write your pallas kernel here")


# speedup vs baseline: 1.0317x; 1.0317x over previous
import functools

import jax
import jax.numpy as jnp
from jax import lax
from jax.experimental import pallas as pl
from jax.experimental.pallas import tpu as pltpu
from jax.experimental.pallas import tpu_sc as plsc

_NUM_Q = 8
_K = 8192
_D = 256
_COMMIT = 0.25
_N = 16 * 1024

_R_BLK = 2048
_C_BLK = 512
_LANES = 128

_NUM_CHUNKS = 4
_CHUNK_W = _K // _NUM_CHUNKS
_BLKS_PER_CHUNK = _CHUNK_W // _C_BLK
_NUM_CBLKS = _K // _C_BLK
_TILES_PER_BLK = _C_BLK // _LANES


def _dist_argmin_body(fn_ref, et_ref, idx_ref,
                      rv0, rt0, rv1, rt1, rv2, rt2, rv3, rt3):
    ci = pl.program_id(1)

    @pl.when(ci == 0)
    def _init():
        for rv, rt in ((rv0, rt0), (rv1, rt1), (rv2, rt2), (rv3, rt3)):
            rv[...] = jnp.full((_R_BLK, _LANES), jnp.inf, jnp.float32)
            rt[...] = jnp.zeros((_R_BLK, _LANES), jnp.int32)

    s = jnp.dot(fn_ref[...], et_ref[...], preferred_element_type=jnp.float32)
    b_in_chunk = lax.rem(ci, _BLKS_PER_CHUNK)
    chunk = lax.div(ci, _BLKS_PER_CHUNK)

    def update(rv_ref, rt_ref):
        rv = rv_ref[...]
        rt = rt_ref[...]
        for j in range(_TILES_PER_BLK):
            d = 1.0 - s[:, j * _LANES:(j + 1) * _LANES]
            t = b_in_chunk * _TILES_PER_BLK + j
            better = d < rv
            rt = jnp.where(better, t, rt)
            rv = jnp.where(better, d, rv)
        rv_ref[...] = rv
        rt_ref[...] = rt

    for c, (rv_ref, rt_ref) in enumerate(
            ((rv0, rt0), (rv1, rt1), (rv2, rt2), (rv3, rt3))):
        @pl.when(chunk == c)
        def _u(rv_ref=rv_ref, rt_ref=rt_ref):
            update(rv_ref, rt_ref)

    @pl.when(ci == pl.num_programs(1) - 1)
    def _emit():
        lane = lax.broadcasted_iota(jnp.int32, (_R_BLK, _LANES), 1)

        def resolve(rv_ref, rt_ref, start):
            rv = rv_ref[...]
            cols = rt_ref[...] * _LANES + lane
            m = jnp.min(rv, axis=1)
            idx = jnp.min(jnp.where(rv == m[:, None], cols, _CHUNK_W), axis=1)
            return m, idx + start

        acc_v, acc_i = None, None
        for c, (rv_ref, rt_ref) in enumerate(
                ((rv0, rt0), (rv1, rt1), (rv2, rt2), (rv3, rt3))):
            m, i = resolve(rv_ref, rt_ref, c * _CHUNK_W)
            mb = m.astype(jnp.bfloat16).astype(jnp.float32)
            if acc_v is None:
                acc_v, acc_i = mb, i
            else:
                take = m < acc_v
                acc_i = jnp.where(take, i, acc_i)
                acc_v = jnp.where(take, mb, acc_v)
        idx_ref[0, 0, :] = acc_i


def _dist_argmin(fn, et):
    nrb = _N // _R_BLK
    out = pl.pallas_call(
        _dist_argmin_body,
        grid=(nrb, _NUM_CBLKS),
        in_specs=[
            pl.BlockSpec((_R_BLK, _D), lambda ri, ci: (ri, 0)),
            pl.BlockSpec((_D, _C_BLK), lambda ri, ci: (0, ci)),
        ],
        out_specs=pl.BlockSpec((1, 1, _R_BLK), lambda ri, ci: (ri, 0, 0)),
        out_shape=jax.ShapeDtypeStruct((nrb, 1, _R_BLK), jnp.int32),
        scratch_shapes=[
            pltpu.VMEM((_R_BLK, _LANES), jnp.float32),
            pltpu.VMEM((_R_BLK, _LANES), jnp.int32),
            pltpu.VMEM((_R_BLK, _LANES), jnp.float32),
            pltpu.VMEM((_R_BLK, _LANES), jnp.int32),
            pltpu.VMEM((_R_BLK, _LANES), jnp.float32),
            pltpu.VMEM((_R_BLK, _LANES), jnp.int32),
            pltpu.VMEM((_R_BLK, _LANES), jnp.float32),
            pltpu.VMEM((_R_BLK, _LANES), jnp.int32),
        ],
    )(fn, et)
    return out.reshape(_N)


def _sc_gather(table, idx):
    info = plsc.get_sparse_core_info()
    nw = info.num_cores * info.num_subcores
    b_per_w = _N // nw
    ch = 128
    n_ch = b_per_w // ch
    mesh = plsc.VectorSubcoreMesh(core_axis_name="c", subcore_axis_name="s")

    @functools.partial(
        pl.kernel,
        mesh=mesh,
        out_type=jax.ShapeDtypeStruct((_N, _D), jnp.float32),
        scratch_types=[
            pltpu.VMEM((ch,), jnp.int32),
            pltpu.VMEM((ch, _D), jnp.float32),
            pltpu.SemaphoreType.DMA,
        ],
    )
    def gk(table_hbm, idx_hbm, out_hbm, idx_v, rows_v, sem):
        wid = lax.axis_index("s") * info.num_cores + lax.axis_index("c")
        base = wid * b_per_w

        def body(cc, carry):
            off = base + cc * ch
            pltpu.sync_copy(idx_hbm.at[pl.ds(off, ch)], idx_v)
            pltpu.async_copy(table_hbm.at[idx_v], rows_v, sem).wait()
            pltpu.sync_copy(rows_v, out_hbm.at[pl.ds(off, ch)])
            return carry

        lax.fori_loop(0, n_ch, body, 0)

    return gk(table, idx)


def _l2n(x):
    n = jnp.sqrt(jnp.sum(x * x, axis=-1, keepdims=True))
    return x / jnp.maximum(n, 1e-12)


def kernel(inputs, codebooks):
    shape = inputs.shape
    et = _l2n(codebooks).transpose(0, 2, 1)
    quantized = jnp.zeros_like(inputs)
    residual = inputs
    losses = []
    idx_list = []
    for i in range(_NUM_Q):
        fn = _l2n(residual.reshape(-1, _D))
        idx = _dist_argmin(fn, et[i])
        q = _sc_gather(codebooks[i], idx).reshape(shape)
        e = jnp.mean((q - residual) ** 2)
        loss = e + _COMMIT * e
        q_st = residual + (q - residual)
        quantized = quantized + q_st
        residual = residual - q_st
        losses.append(loss)
        idx_list.append(idx[:, None])
    total_vq_loss = jnp.mean(jnp.stack(losses))
    indices = jnp.stack(idx_list)
    return (quantized, total_vq_loss, indices)
